# Initial kernel scaffold; baseline (speedup 1.0000x reference)
import jax
import jax.numpy as jnp
from jax.experimental import pallas as pl


def kernel(sequence, tag, item_emb_table, ln_weight, ln_bias):
    raise NotImplementedError("write your pallas kernel here")


# speedup vs baseline: 1.7456x; 1.7456x over previous
import jax
import jax.numpy as jnp
from jax import lax
from jax.experimental import pallas as pl
from jax.experimental.pallas import tpu as pltpu
from jax.experimental.pallas import tpu_sc as plsc

_B, _L, _D = 4096, 200, 64
_EPS = 1e-12
_NC, _NS = 2, 16
_NW = _NC * _NS
_BPW = _B // _NW
_NFULL = _L // 16
_TAIL_OFF = _L - 16


def _rsqrt(v):
    vi = lax.bitcast_convert_type(v, jnp.int32)
    yi = jnp.int32(0x5F3759DF) - lax.shift_right_arithmetic(vi, jnp.int32(1))
    y = lax.bitcast_convert_type(yi, jnp.float32)
    for _ in range(3):
        y = y * (1.5 - 0.5 * v * y * y)
    return y


def _body(seq_ref, tag_ref, tab_ref, w_ref, b_ref, out_ref,
          idx_v, tagidx_v, rows0, rows1, tagrows_v, out_v, wb_v,
          sem0, sem1, semt):
    wid = lax.axis_index("s") * _NC + lax.axis_index("c")
    base = wid * _BPW

    pltpu.sync_copy(seq_ref.at[pl.ds(base, _BPW)], idx_v)
    pltpu.sync_copy(tag_ref.at[pl.ds(base, _BPW)], tagidx_v)
    pltpu.sync_copy(w_ref, wb_v.at[0])
    pltpu.sync_copy(b_ref, wb_v.at[1])
    pltpu.make_async_copy(tab_ref.at[tagidx_v], tagrows_v, semt).start()
    pltpu.make_async_copy(tab_ref.at[idx_v.at[0]], rows0, sem0).start()
    pltpu.make_async_copy(tab_ref.at[idx_v.at[1]], rows1, sem1).start()
    pltpu.make_async_copy(tab_ref.at[tagidx_v], tagrows_v, semt).wait()

    lane = lax.broadcasted_iota(jnp.int32, (16,), 0)

    def ln_chunk(rows, l0, mvf, carry):
        acc0, acc1, acc2, acc3, msum = carry
        msum = msum + mvf
        for k in range(16):
            l = l0 + k
            x0 = rows[l, pl.ds(0, 16)]
            x1 = rows[l, pl.ds(16, 16)]
            x2 = rows[l, pl.ds(32, 16)]
            x3 = rows[l, pl.ds(48, 16)]
            s = (x0 + x1) + (x2 + x3)
            q = (x0 * x0 + x1 * x1) + (x2 * x2 + x3 * x3)
            u = jnp.sum(s) * (1.0 / 64.0)
            ex2 = jnp.sum(q) * (1.0 / 64.0)
            var = jnp.maximum(ex2 - u * u, 0.0)
            rm = _rsqrt(var + _EPS) * mvf[k]
            urm = u * rm
            rmv = jnp.broadcast_to(rm, (16,))
            urmv = jnp.broadcast_to(urm, (16,))
            acc0 = acc0 + (x0 * rmv - urmv)
            acc1 = acc1 + (x1 * rmv - urmv)
            acc2 = acc2 + (x2 * rmv - urmv)
            acc3 = acc3 + (x3 * rmv - urmv)
        return acc0, acc1, acc2, acc3, msum

    def process(g, rows, sem):
        pltpu.make_async_copy(tab_ref.at[idx_v.at[g]], rows, sem).wait()

        z = jnp.zeros((16,), jnp.float32)

        def cbody(c, carry):
            l0 = c * 16
            mvf = (idx_v[g, pl.ds(l0, 16)] > 0).astype(jnp.float32)
            return ln_chunk(rows, l0, mvf, carry)

        carry = lax.fori_loop(0, _NFULL, cbody, (z, z, z, z, z))
        mvt = ((idx_v[g, pl.ds(_TAIL_OFF, 16)] > 0) & (lane >= 8)).astype(
            jnp.float32)
        acc0, acc1, acc2, acc3, msum = ln_chunk(rows, _TAIL_OFF, mvt, carry)

        nvalid = jnp.sum(msum)
        denv = jnp.broadcast_to(jnp.maximum(nvalid, 1.0), (16,))
        rdv = 1.0 / denv
        tvv = jnp.broadcast_to(jnp.minimum(nvalid, 1.0), (16,))
        for f, acc in enumerate((acc0, acc1, acc2, acc3)):
            wf = wb_v[0, pl.ds(16 * f, 16)]
            bf = wb_v[1, pl.ds(16 * f, 16)]
            tg = tagrows_v[g, pl.ds(16 * f, 16)]
            out_v[g, pl.ds(16 * f, 16)] = acc * rdv * wf + bf * tvv + tg

        @pl.when(g + 2 < _BPW)
        def _():
            pltpu.make_async_copy(tab_ref.at[idx_v.at[g + 2]], rows, sem).start()

    def pair(i, c):
        process(2 * i, rows0, sem0)
        process(2 * i + 1, rows1, sem1)
        return c

    lax.fori_loop(0, _BPW // 2, pair, 0)
    pltpu.sync_copy(out_v, out_ref.at[pl.ds(base, _BPW)])


def _build():
    return pl.kernel(
        _body,
        out_type=jax.ShapeDtypeStruct((_B, _D), jnp.float32),
        mesh=plsc.VectorSubcoreMesh(
            core_axis_name="c", subcore_axis_name="s",
            num_cores=_NC, num_subcores=_NS),
        scratch_types=[
            pltpu.VMEM((_BPW, _L), jnp.int32),
            pltpu.VMEM((_BPW,), jnp.int32),
            pltpu.VMEM((_L, _D), jnp.float32),
            pltpu.VMEM((_L, _D), jnp.float32),
            pltpu.VMEM((_BPW, _D), jnp.float32),
            pltpu.VMEM((_BPW, _D), jnp.float32),
            pltpu.VMEM((2, _D), jnp.float32),
            pltpu.SemaphoreType.DMA,
            pltpu.SemaphoreType.DMA,
            pltpu.SemaphoreType.DMA,
        ],
        compiler_params=pltpu.CompilerParams(use_tc_tiling_on_sc=False,
                                             needs_layout_passes=False),
    )


def kernel(sequence, tag, item_emb_table, ln_weight, ln_bias):
    return _build()(sequence, tag[:, 0], item_emb_table, ln_weight, ln_bias)
